# baseline (speedup 1.0000x reference)
import math

import numpy as np
import jax
import jax.numpy as jnp
from jax import lax
from jax.experimental import pallas as pl
from jax.experimental.pallas import tpu as pltpu
from jax.experimental.pallas import tpu_sc as plsc

N_NODES = 10000
N_EDGES = 160000
N_BASIS = 8
R_MAX = 5.0
AVG_NEIGH = 16.0
EDGE_OH = 128
LATENT = 128
MUL = 32
IR_DIMS = (1, 3, 5)
SH_DIM = 9
N_IR = 3
EPS = 1e-08
F_DIM = MUL * sum(IR_DIMS)

BE = 3200
BN = 2000


def _expansion_mats():
    R = np.zeros((MUL * N_IR, F_DIM), np.float32)
    S = np.zeros((SH_DIM, F_DIM), np.float32)
    off = 0
    shoff = 0
    for i, d in enumerate(IR_DIMS):
        for m in range(MUL):
            for j in range(d):
                R[i * MUL + m, off + m * d + j] = 1.0
                S[shoff + j, off + m * d + j] = 1.0
        off += MUL * d
        shoff += d
    return R, S

_R_NP, _S_NP = _expansion_mats()


_TDOT = (((0,), (0,)), ((), ()))


def _edge_body(len_ref, oh_ref, sht_ref, bw_ref, w0_ref, w1_ref, w2_ref,
               we_ref, r_ref, s_ref, raw_ref, p0_ref, p1_ref, p2_ref,
               eft_ref):
    xs = len_ref[...]
    w = bw_ref[...]
    sins = jnp.sin(w * (xs * (1.0 / R_MAX)))
    invt = (2.0 / R_MAX) * sins / xs
    s0 = 1.0 / math.sqrt(EDGE_OH + N_BASIS)
    s1 = 1.0 / math.sqrt(LATENT)
    h = (oh_ref[...] @ w0_ref[0:EDGE_OH, :]
         + lax.dot_general(invt, w0_ref[EDGE_OH:, :], _TDOT))
    h = jax.nn.silu(h * s0)
    h = jax.nn.silu((h @ w1_ref[...]) * s1)
    raw = (h @ w2_ref[...]) * s1
    raw_ref[...] = raw
    wcomb = (we_ref[...] * s1) @ r_ref[...]
    ef = (raw @ wcomb) * lax.dot_general(sht_ref[...], s_ref[...], _TDOT)
    p0_ref[...] = ef[:, 0:128]
    p1_ref[...] = ef[:, 128:256]
    p2_ref[...] = ef[:, 256:288]
    wcombt = lax.dot_general(r_ref[...], we_ref[...] * s1,
                             (((0,), (1,)), ((), ())))
    shst = lax.dot_general(s_ref[...], sht_ref[...], _TDOT)
    eft_ref[...] = lax.dot_general(wcombt, raw,
                                   (((1,), (1,)), ((), ()))) * shst


def _edge_pipeline_half(half, prev, edge_length, edge_one_hot, edge_sh_t,
                        bessel_w, tb_w0, tb_w1, tb_w2, env_w, R, S):
    n_blocks = EHALF // BE
    off = half * n_blocks
    full = lambda shape: pl.BlockSpec(shape, lambda i: (0, 0))
    anyspec = pl.BlockSpec(memory_space=pl.ANY)
    n_prev = len(prev) if prev else 0

    def body(*refs):
        ins = refs[:10]
        raw_ref, eft_ref, p0_ref, p1_ref, p2_ref = refs[10 + n_prev:]
        _edge_body(*ins, raw_ref, p0_ref, p1_ref, p2_ref, eft_ref)

    return pl.pallas_call(
        body,
        grid=(n_blocks,),
        in_specs=[
            pl.BlockSpec((1, BE), lambda i: (0, i + off)),
            pl.BlockSpec((BE, EDGE_OH), lambda i: (i + off, 0)),
            pl.BlockSpec((SH_DIM, BE), lambda i: (0, i + off)),
            full((N_BASIS, 1)),
            full((EDGE_OH + N_BASIS, LATENT)),
            full((LATENT, LATENT)),
            full((LATENT, LATENT)),
            full((LATENT, MUL * N_IR)),
            full((MUL * N_IR, F_DIM)),
            full((SH_DIM, F_DIM)),
        ] + [anyspec] * n_prev,
        out_specs=[
            pl.BlockSpec((BE, LATENT), lambda i: (i + off, 0)),
            pl.BlockSpec((F_DIM, BE), lambda i: (0, i + off)),
            pl.BlockSpec((BE, 128), lambda i: (i, 0)),
            pl.BlockSpec((BE, 128), lambda i: (i, 0)),
            pl.BlockSpec((BE, 32), lambda i: (i, 0)),
        ],
        out_shape=[
            jax.ShapeDtypeStruct((N_EDGES, LATENT), jnp.float32),
            jax.ShapeDtypeStruct((F_DIM, N_EDGES), jnp.float32),
            jax.ShapeDtypeStruct((EHALF, 128), jnp.float32),
            jax.ShapeDtypeStruct((EHALF, 128), jnp.float32),
            jax.ShapeDtypeStruct((EHALF, 32), jnp.float32),
        ],
        input_output_aliases={10 + i: i for i in range(n_prev)},
    )(edge_length.reshape(1, N_EDGES), edge_one_hot, edge_sh_t,
      bessel_w.reshape(N_BASIS, 1), tb_w0, tb_w1, tb_w2, env_w, R, S,
      *(prev or ()))


def _sln_body(*refs):
    xs = refs[0:NHALF]
    xbs = refs[NHALF:2 * NHALF]
    lnw_ref, lnb_ref, r_ref, out_ref = refs[2 * NHALF:]
    x = xs[0][...]
    for r in xs[1:]:
        x = x + r[...]
    xb = xbs[0][...]
    for r in xbs[1:]:
        xb = xb + r[...]
    x = jnp.concatenate([x[:, 0:256], x[:, 256:F_DIM] + xb], axis=1)
    x = x * (1.0 / math.sqrt(AVG_NEIGH))
    col = lax.broadcasted_iota(jnp.int32, (1, F_DIM), 1)
    m0mask = (col < MUL).astype(jnp.float32)
    m0 = jnp.sum(x * m0mask, axis=1, keepdims=True) * (1.0 / MUL)
    xc = x - m0 * m0mask
    vw = jnp.where(col < MUL, 1.0 / (N_IR * MUL * 1),
                   jnp.where(col < MUL * 4, 1.0 / (N_IR * MUL * 3),
                             1.0 / (N_IR * MUL * 5))).astype(jnp.float32)
    var = jnp.sum(xc * xc * vw, axis=1, keepdims=True)
    inv = lax.rsqrt(var + EPS)
    wcol = lnw_ref[...] @ r_ref[...]
    bcol = lnb_ref[...] @ r_ref[0:MUL, :]
    out_ref[...] = xc * inv * wcol + bcol


def _sln(nss, nsbs, ln_w, ln_b, R):
    n_blocks = N_NODES // BN
    return pl.pallas_call(
        _sln_body,
        grid=(n_blocks,),
        in_specs=(
            [pl.BlockSpec((BN, F_DIM), lambda i: (i, 0))] * NHALF
            + [pl.BlockSpec((BN, 32), lambda i: (i, 0))] * NHALF
            + [
                pl.BlockSpec((1, MUL * N_IR), lambda i: (0, 0)),
                pl.BlockSpec((1, MUL), lambda i: (0, 0)),
                pl.BlockSpec((MUL * N_IR, F_DIM), lambda i: (0, 0)),
            ]
        ),
        out_specs=pl.BlockSpec((BN, F_DIM), lambda i: (i, 0)),
        out_shape=jax.ShapeDtypeStruct((N_NODES, F_DIM), jnp.float32),
    )(*nss, *nsbs, ln_w.reshape(1, MUL * N_IR), ln_b.reshape(1, MUL), R)


NHALF = 5
EHALF = N_EDGES // NHALF
COLH = F_DIM // 2
EPT = EHALF // 16
W = 50
NCH = EPT // W
NRT = N_NODES // 16
ZCH = 25
NRC = NRT // ZCH


def _scatter_body(p0_hbm, p1_hbm, p2_hbm, ec_hbm, out_hbm, out2_hbm, idx_v,
                  buf_a, buf_b, buf_c, buf_d, sem_a, sem_b, sem_c, sem_d,
                  acc_a, acc_b):
    c = lax.axis_index("c")
    s = lax.axis_index("s")

    def _zrow(j, _):
        def _zcol(k, _):
            buf_a[j, pl.ds(k * 16, 16)] = jnp.zeros((16,), jnp.float32)
            return 0
        return lax.fori_loop(0, 128 // 16, _zcol, 0)
    lax.fori_loop(0, ZCH, _zrow, 0)
    for k in range(NRC):
        r0 = s * NRT + k * ZCH
        pltpu.sync_copy(buf_a.at[pl.ds(0, ZCH)], acc_a.at[pl.ds(r0, ZCH)])
        pltpu.sync_copy(buf_a.at[pl.ds(0, ZCH), pl.ds(0, 32)],
                        acc_b.at[pl.ds(r0, ZCH)])

    pltpu.sync_copy(ec_hbm.at[pl.ds(s * NCH, NCH)], idx_v)
    plsc.subcore_barrier()

    hw = NCH // 2
    p2o = c * hw

    def _nsrc(j):
        return p2_hbm.at[pl.ds(s * EPT + j * W, W)]

    def _mainloop(piece_hbm):
        def _msrc(j):
            return piece_hbm.at[pl.ds(s * EPT + j * W, W)]
        pltpu.async_copy(_msrc(0), buf_a, sem_a)
        pltpu.async_copy(_nsrc(p2o), buf_c, sem_c)

        def _pair(p, _):
            j = p * 2
            pltpu.make_async_copy(_msrc(j), buf_a, sem_a).wait()
            pltpu.async_copy(_msrc(j + 1), buf_b, sem_b)
            pltpu.sync_copy(buf_a, acc_a.at[idx_v.at[j]], add=True)

            @pl.when(j < hw)
            def _():
                jn = p2o + j
                pltpu.make_async_copy(_nsrc(jn), buf_c, sem_c).wait()
                pltpu.async_copy(_nsrc(jn + 1), buf_d, sem_d)
                pltpu.sync_copy(buf_c, acc_b.at[idx_v.at[jn]], add=True)
                pltpu.make_async_copy(_nsrc(jn + 1), buf_d, sem_d).wait()

                @pl.when(j + 2 < hw)
                def _():
                    pltpu.async_copy(_nsrc(jn + 2), buf_c, sem_c)
                pltpu.sync_copy(buf_d, acc_b.at[idx_v.at[jn + 1]], add=True)

            pltpu.make_async_copy(_msrc(j + 1), buf_b, sem_b).wait()

            @pl.when(j + 2 < NCH)
            def _():
                pltpu.async_copy(_msrc(j + 2), buf_a, sem_a)
            pltpu.sync_copy(buf_b, acc_a.at[idx_v.at[j + 1]], add=True)
            return 0
        lax.fori_loop(0, NCH // 2, _pair, 0)

    @pl.when(c == 0)
    def _():
        _mainloop(p0_hbm)

    @pl.when(c == 1)
    def _():
        _mainloop(p1_hbm)
    plsc.subcore_barrier()

    for k in range(NRC):
        r0 = s * NRT + k * ZCH
        pltpu.sync_copy(acc_a.at[pl.ds(r0, ZCH)], buf_a.at[pl.ds(0, ZCH)])
        pltpu.sync_copy(acc_b.at[pl.ds(r0, ZCH)], buf_c.at[pl.ds(0, ZCH)])

        @pl.when(c == 0)
        def _():
            pltpu.sync_copy(buf_a.at[pl.ds(0, ZCH)],
                            out_hbm.at[pl.ds(r0, ZCH), pl.ds(0, 128)])
            pltpu.sync_copy(buf_c.at[pl.ds(0, ZCH)],
                            out_hbm.at[pl.ds(r0, ZCH), pl.ds(256, 32)])

        @pl.when(c == 1)
        def _():
            pltpu.sync_copy(buf_a.at[pl.ds(0, ZCH)],
                            out_hbm.at[pl.ds(r0, ZCH), pl.ds(128, 128)])
            pltpu.sync_copy(buf_c.at[pl.ds(0, ZCH)],
                            out2_hbm.at[pl.ds(r0, ZCH)])


def _sc_scatter(p0, p1, p2, edge_center2d):
    return pl.kernel(
        _scatter_body,
        out_type=[jax.ShapeDtypeStruct((N_NODES, F_DIM), jnp.float32),
                  jax.ShapeDtypeStruct((N_NODES, 32), jnp.float32)],
        mesh=plsc.VectorSubcoreMesh(core_axis_name="c", subcore_axis_name="s"),
        scratch_types=[
            pltpu.VMEM((NCH, W), jnp.int32),
            pltpu.VMEM((W, 128), jnp.float32),
            pltpu.VMEM((W, 128), jnp.float32),
            pltpu.VMEM((W, 32), jnp.float32),
            pltpu.VMEM((W, 32), jnp.float32),
            pltpu.SemaphoreType.DMA,
            pltpu.SemaphoreType.DMA,
            pltpu.SemaphoreType.DMA,
            pltpu.SemaphoreType.DMA,
            pltpu.VMEM_SHARED((N_NODES, 128), jnp.float32),
            pltpu.VMEM_SHARED((N_NODES, 32), jnp.float32),
        ],
        compiler_params=pltpu.CompilerParams(use_tc_tiling_on_sc=False),
    )(p0, p1, p2, edge_center2d)


def kernel(edge_index, atom_type, edge_sh, edge_length, edge_one_hot,
           bessel_w, tb_w0, tb_w1, tb_w2, env_w, ln_w, ln_b):
    R = jnp.asarray(_R_NP)
    S = jnp.asarray(_S_NP)
    args = (edge_length, edge_one_hot, edge_sh.T, bessel_w,
            tb_w0, tb_w1, tb_w2, env_w, R, S)
    ec = edge_index[0]
    prev = None
    nss, nsbs = [], []
    for h in range(NHALF):
        o = _edge_pipeline_half(h, prev, *args)
        prev = o[:2]
        ns, nsb = _sc_scatter(
            o[2], o[3], o[4],
            ec[h * EHALF:(h + 1) * EHALF].reshape(16 * NCH, W))
        nss.append(ns)
        nsbs.append(nsb)
    raw_latents, ef_t = prev
    edge_features = ef_t.T
    node_features = _sln(nss, nsbs, ln_w, ln_b, R)
    return (raw_latents, node_features, edge_features)


# speedup vs baseline: 1.1070x; 1.1070x over previous
import math

import numpy as np
import jax
import jax.numpy as jnp
from jax import lax
from jax.experimental import pallas as pl
from jax.experimental.pallas import tpu as pltpu
from jax.experimental.pallas import tpu_sc as plsc

N_NODES = 10000
N_EDGES = 160000
N_BASIS = 8
R_MAX = 5.0
AVG_NEIGH = 16.0
EDGE_OH = 128
LATENT = 128
MUL = 32
IR_DIMS = (1, 3, 5)
SH_DIM = 9
N_IR = 3
EPS = 1e-08
F_DIM = MUL * sum(IR_DIMS)

BE = 3200
BN = 2000


def _expansion_mats():
    R = np.zeros((MUL * N_IR, F_DIM), np.float32)
    S = np.zeros((SH_DIM, F_DIM), np.float32)
    off = 0
    shoff = 0
    for i, d in enumerate(IR_DIMS):
        for m in range(MUL):
            for j in range(d):
                R[i * MUL + m, off + m * d + j] = 1.0
                S[shoff + j, off + m * d + j] = 1.0
        off += MUL * d
        shoff += d
    return R, S

_R_NP, _S_NP = _expansion_mats()


_TDOT = (((0,), (0,)), ((), ()))


def _edge_body(len_ref, oh_ref, sht_ref, bw_ref, w0_ref, w1_ref, w2_ref,
               we_ref, r_ref, s_ref, raw_ref, p0_ref, p1_ref, p2_ref,
               eft_ref):
    xs = len_ref[...]
    w = bw_ref[...]
    sins = jnp.sin(w * (xs * (1.0 / R_MAX)))
    invt = (2.0 / R_MAX) * sins / xs
    s0 = 1.0 / math.sqrt(EDGE_OH + N_BASIS)
    s1 = 1.0 / math.sqrt(LATENT)
    h = (oh_ref[...] @ w0_ref[0:EDGE_OH, :]
         + lax.dot_general(invt, w0_ref[EDGE_OH:, :], _TDOT))
    h = jax.nn.silu(h * s0)
    h = jax.nn.silu((h @ w1_ref[...]) * s1)
    raw = (h @ w2_ref[...]) * s1
    raw_ref[...] = raw
    wcomb = (we_ref[...] * s1) @ r_ref[...]
    ef = (raw @ wcomb) * lax.dot_general(sht_ref[...], s_ref[...], _TDOT)
    p0_ref[...] = ef[:, 0:128]
    p1_ref[...] = ef[:, 128:256]
    p2_ref[...] = ef[:, 256:288]
    wcombt = lax.dot_general(r_ref[...], we_ref[...] * s1,
                             (((0,), (1,)), ((), ())))
    shst = lax.dot_general(s_ref[...], sht_ref[...], _TDOT)
    eft_ref[...] = lax.dot_general(wcombt, raw,
                                   (((1,), (1,)), ((), ()))) * shst


def _edge_pipeline_half(half, prev, edge_length, edge_one_hot, edge_sh_t,
                        bessel_w, tb_w0, tb_w1, tb_w2, env_w, R, S):
    n_blocks = EHALF // BE
    off = half * n_blocks
    full = lambda shape: pl.BlockSpec(shape, lambda i: (0, 0))
    anyspec = pl.BlockSpec(memory_space=pl.ANY)
    n_prev = len(prev) if prev else 0

    def body(*refs):
        ins = refs[:10]
        raw_ref, eft_ref, p0_ref, p1_ref, p2_ref = refs[10 + n_prev:]
        _edge_body(*ins, raw_ref, p0_ref, p1_ref, p2_ref, eft_ref)

    return pl.pallas_call(
        body,
        grid=(n_blocks,),
        in_specs=[
            pl.BlockSpec((1, BE), lambda i: (0, i + off)),
            pl.BlockSpec((BE, EDGE_OH), lambda i: (i + off, 0)),
            pl.BlockSpec((SH_DIM, BE), lambda i: (0, i + off)),
            full((N_BASIS, 1)),
            full((EDGE_OH + N_BASIS, LATENT)),
            full((LATENT, LATENT)),
            full((LATENT, LATENT)),
            full((LATENT, MUL * N_IR)),
            full((MUL * N_IR, F_DIM)),
            full((SH_DIM, F_DIM)),
        ] + [anyspec] * n_prev,
        out_specs=[
            pl.BlockSpec((BE, LATENT), lambda i: (i + off, 0)),
            pl.BlockSpec((F_DIM, BE), lambda i: (0, i + off)),
            pl.BlockSpec((BE, 128), lambda i: (i, 0)),
            pl.BlockSpec((BE, 128), lambda i: (i, 0)),
            pl.BlockSpec((BE, 32), lambda i: (i, 0)),
        ],
        out_shape=[
            jax.ShapeDtypeStruct((N_EDGES, LATENT), jnp.float32),
            jax.ShapeDtypeStruct((F_DIM, N_EDGES), jnp.float32),
            jax.ShapeDtypeStruct((EHALF, 128), jnp.float32),
            jax.ShapeDtypeStruct((EHALF, 128), jnp.float32),
            jax.ShapeDtypeStruct((EHALF, 32), jnp.float32),
        ],
        input_output_aliases={10 + i: i for i in range(n_prev)},
    )(edge_length.reshape(1, N_EDGES), edge_one_hot, edge_sh_t,
      bessel_w.reshape(N_BASIS, 1), tb_w0, tb_w1, tb_w2, env_w, R, S,
      *(prev or ()))


def _sln_body(*refs):
    xs = refs[0:NHALF]
    xbs = refs[NHALF:2 * NHALF]
    lnw_ref, lnb_ref, r_ref, out_ref = refs[2 * NHALF:]
    x = xs[0][...]
    for r in xs[1:]:
        x = x + r[...]
    xb = xbs[0][...]
    for r in xbs[1:]:
        xb = xb + r[...]
    x = jnp.concatenate([x[:, 0:256], x[:, 256:F_DIM] + xb], axis=1)
    x = x * (1.0 / math.sqrt(AVG_NEIGH))
    col = lax.broadcasted_iota(jnp.int32, (1, F_DIM), 1)
    m0mask = (col < MUL).astype(jnp.float32)
    m0 = jnp.sum(x * m0mask, axis=1, keepdims=True) * (1.0 / MUL)
    xc = x - m0 * m0mask
    vw = jnp.where(col < MUL, 1.0 / (N_IR * MUL * 1),
                   jnp.where(col < MUL * 4, 1.0 / (N_IR * MUL * 3),
                             1.0 / (N_IR * MUL * 5))).astype(jnp.float32)
    var = jnp.sum(xc * xc * vw, axis=1, keepdims=True)
    inv = lax.rsqrt(var + EPS)
    wcol = lnw_ref[...] @ r_ref[...]
    bcol = lnb_ref[...] @ r_ref[0:MUL, :]
    out_ref[...] = xc * inv * wcol + bcol


def _sln(nss, nsbs, ln_w, ln_b, R):
    n_blocks = N_NODES // BN
    return pl.pallas_call(
        _sln_body,
        grid=(n_blocks,),
        in_specs=(
            [pl.BlockSpec((BN, F_DIM), lambda i: (i, 0))] * NHALF
            + [pl.BlockSpec((BN, 32), lambda i: (i, 0))] * NHALF
            + [
                pl.BlockSpec((1, MUL * N_IR), lambda i: (0, 0)),
                pl.BlockSpec((1, MUL), lambda i: (0, 0)),
                pl.BlockSpec((MUL * N_IR, F_DIM), lambda i: (0, 0)),
            ]
        ),
        out_specs=pl.BlockSpec((BN, F_DIM), lambda i: (i, 0)),
        out_shape=jax.ShapeDtypeStruct((N_NODES, F_DIM), jnp.float32),
    )(*nss, *nsbs, ln_w.reshape(1, MUL * N_IR), ln_b.reshape(1, MUL), R)


NHALF = 2
EHALF = N_EDGES // NHALF
COLH = F_DIM // 2
EPT = EHALF // 16
W = 50
NCH = EPT // W
NRT = N_NODES // 16
ZCH = 25
NRC = NRT // ZCH


def _scatter_body(p0_hbm, p1_hbm, p2_hbm, ec_hbm, out_hbm, out2_hbm, idx_v,
                  buf_a, buf_b, buf_c, buf_d, sem_a, sem_b, sem_c, sem_d,
                  acc_a, acc_b):
    c = lax.axis_index("c")
    s = lax.axis_index("s")

    def _zrow(j, _):
        def _zcol(k, _):
            buf_a[j, pl.ds(k * 16, 16)] = jnp.zeros((16,), jnp.float32)
            return 0
        return lax.fori_loop(0, 128 // 16, _zcol, 0)
    lax.fori_loop(0, ZCH, _zrow, 0)
    for k in range(NRC):
        r0 = s * NRT + k * ZCH
        pltpu.sync_copy(buf_a.at[pl.ds(0, ZCH)], acc_a.at[pl.ds(r0, ZCH)])
        pltpu.sync_copy(buf_a.at[pl.ds(0, ZCH), pl.ds(0, 32)],
                        acc_b.at[pl.ds(r0, ZCH)])

    pltpu.sync_copy(ec_hbm.at[pl.ds(s * NCH, NCH)], idx_v)
    plsc.subcore_barrier()

    hw = NCH // 2
    p2o = c * hw

    def _nsrc(j):
        return p2_hbm.at[pl.ds(s * EPT + j * W, W)]

    def _mainloop(piece_hbm):
        def _msrc(j):
            return piece_hbm.at[pl.ds(s * EPT + j * W, W)]
        pltpu.async_copy(_msrc(0), buf_a, sem_a)
        pltpu.async_copy(_nsrc(p2o), buf_c, sem_c)

        def _pair(p, _):
            j = p * 2
            pltpu.make_async_copy(_msrc(j), buf_a, sem_a).wait()
            pltpu.async_copy(_msrc(j + 1), buf_b, sem_b)
            pltpu.sync_copy(buf_a, acc_a.at[idx_v.at[j]], add=True)

            @pl.when(j < hw)
            def _():
                jn = p2o + j
                pltpu.make_async_copy(_nsrc(jn), buf_c, sem_c).wait()
                pltpu.async_copy(_nsrc(jn + 1), buf_d, sem_d)
                pltpu.sync_copy(buf_c, acc_b.at[idx_v.at[jn]], add=True)
                pltpu.make_async_copy(_nsrc(jn + 1), buf_d, sem_d).wait()

                @pl.when(j + 2 < hw)
                def _():
                    pltpu.async_copy(_nsrc(jn + 2), buf_c, sem_c)
                pltpu.sync_copy(buf_d, acc_b.at[idx_v.at[jn + 1]], add=True)

            pltpu.make_async_copy(_msrc(j + 1), buf_b, sem_b).wait()

            @pl.when(j + 2 < NCH)
            def _():
                pltpu.async_copy(_msrc(j + 2), buf_a, sem_a)
            pltpu.sync_copy(buf_b, acc_a.at[idx_v.at[j + 1]], add=True)
            return 0
        lax.fori_loop(0, NCH // 2, _pair, 0)

    @pl.when(c == 0)
    def _():
        _mainloop(p0_hbm)

    @pl.when(c == 1)
    def _():
        _mainloop(p1_hbm)
    plsc.subcore_barrier()

    for k in range(NRC):
        r0 = s * NRT + k * ZCH
        pltpu.sync_copy(acc_a.at[pl.ds(r0, ZCH)], buf_a.at[pl.ds(0, ZCH)])
        pltpu.sync_copy(acc_b.at[pl.ds(r0, ZCH)], buf_c.at[pl.ds(0, ZCH)])

        @pl.when(c == 0)
        def _():
            pltpu.sync_copy(buf_a.at[pl.ds(0, ZCH)],
                            out_hbm.at[pl.ds(r0, ZCH), pl.ds(0, 128)])
            pltpu.sync_copy(buf_c.at[pl.ds(0, ZCH)],
                            out_hbm.at[pl.ds(r0, ZCH), pl.ds(256, 32)])

        @pl.when(c == 1)
        def _():
            pltpu.sync_copy(buf_a.at[pl.ds(0, ZCH)],
                            out_hbm.at[pl.ds(r0, ZCH), pl.ds(128, 128)])
            pltpu.sync_copy(buf_c.at[pl.ds(0, ZCH)],
                            out2_hbm.at[pl.ds(r0, ZCH)])


def _sc_scatter(p0, p1, p2, edge_center2d):
    return pl.kernel(
        _scatter_body,
        out_type=[jax.ShapeDtypeStruct((N_NODES, F_DIM), jnp.float32),
                  jax.ShapeDtypeStruct((N_NODES, 32), jnp.float32)],
        mesh=plsc.VectorSubcoreMesh(core_axis_name="c", subcore_axis_name="s"),
        scratch_types=[
            pltpu.VMEM((NCH, W), jnp.int32),
            pltpu.VMEM((W, 128), jnp.float32),
            pltpu.VMEM((W, 128), jnp.float32),
            pltpu.VMEM((W, 32), jnp.float32),
            pltpu.VMEM((W, 32), jnp.float32),
            pltpu.SemaphoreType.DMA,
            pltpu.SemaphoreType.DMA,
            pltpu.SemaphoreType.DMA,
            pltpu.SemaphoreType.DMA,
            pltpu.VMEM_SHARED((N_NODES, 128), jnp.float32),
            pltpu.VMEM_SHARED((N_NODES, 32), jnp.float32),
        ],
        compiler_params=pltpu.CompilerParams(use_tc_tiling_on_sc=False),
    )(p0, p1, p2, edge_center2d)


def kernel(edge_index, atom_type, edge_sh, edge_length, edge_one_hot,
           bessel_w, tb_w0, tb_w1, tb_w2, env_w, ln_w, ln_b):
    R = jnp.asarray(_R_NP)
    S = jnp.asarray(_S_NP)
    args = (edge_length, edge_one_hot, edge_sh.T, bessel_w,
            tb_w0, tb_w1, tb_w2, env_w, R, S)
    ec = edge_index[0]
    prev = None
    nss, nsbs = [], []
    for h in range(NHALF):
        o = _edge_pipeline_half(h, prev, *args)
        prev = o[:2]
        ns, nsb = _sc_scatter(
            o[2], o[3], o[4],
            ec[h * EHALF:(h + 1) * EHALF].reshape(16 * NCH, W))
        nss.append(ns)
        nsbs.append(nsb)
    raw_latents, ef_t = prev
    edge_features = ef_t.T
    node_features = _sln(nss, nsbs, ln_w, ln_b, R)
    return (raw_latents, node_features, edge_features)
